# baseline (speedup 1.0000x reference)
import jax
import jax.numpy as jnp
from jax import lax
from jax.experimental import pallas as pl
from jax.experimental.pallas import tpu as pltpu
from jax.experimental.pallas import tpu_sc as plsc

VOCAB = 1000000
EMBED_DIM = 32
BATCH = 4096
HIST = 200

NC = 2
NS = 16
NW = NC * NS

LANE = 128
H8 = HIST // 8
CB = BATCH // LANE
TILES = H8 * CB
TPW = TILES // NW
CPW = TPW * 2


def _body(xr, table_hbm, out_hbm, idx_res, rows0, rows1, dst0, dst1,
          gs0, gs1, ss0, ss1):
    rows_v = (rows0, rows1)
    dst_v = (dst0, dst1)
    gat_s = (gs0, gs1)
    st_s = (ss0, ss1)

    wid = lax.axis_index("s") * NC + lax.axis_index("c")

    pltpu.sync_copy(xr.at[pl.ds(TPW * wid, TPW)], idx_res)

    iota = lax.iota(jnp.int32, 16)

    def fire(g, b):
        i = g // 2
        q = g % 2
        for hh in range(4):
            pltpu.async_copy(
                table_hbm.at[idx_res.at[i, 4 * q + hh]],
                rows_v[b].at[pl.ds(LANE * hh, LANE)],
                gat_s[b],
            )

    def drain_gat(b):
        pltpu.make_async_copy(
            out_hbm.at[pl.ds(0, 16), 0],
            dst_v[b],
            gat_s[b],
        ).wait()

    def out_slice(g):
        t = TPW * wid + g // 2
        h8 = t // CB
        c = lax.rem(t, CB)
        r0 = 32 * h8 + 16 * (g % 2)
        return out_hbm.at[pl.ds(r0, 16), c]

    def transpose(b):
        def tbody(m, carry):
            hh = m // 4
            e4 = lax.rem(m, 4)
            col0 = 8 * e4
            row0 = 128 * hh
            for s in range(8):
                colv = jnp.full((16,), col0 + s, jnp.int32)
                for j in range(8):
                    rowv = row0 + 16 * j + iota
                    v = plsc.load_gather(rows_v[b], [rowv, colv])
                    dst_v[b][m, s, pl.ds(16 * j, 16)] = v
            return carry

        lax.fori_loop(0, 16, tbody, 0)

    def start_store(g, b):
        pltpu.async_copy(dst_v[b], out_slice(g), st_s[b])

    def wait_store(g, b):
        pltpu.make_async_copy(dst_v[b], out_slice(g), st_s[b]).wait()

    def chunk(g, b, fire_next, wait_prev):
        if fire_next:
            fire(g + 1, b ^ 1)
        drain_gat(b)
        if wait_prev:
            wait_store(g - 2, b)
        transpose(b)
        start_store(g, b)

    fire(0, 0)
    chunk(0, 0, True, False)
    chunk(1, 1, True, False)

    def step(it, carry):
        g = 2 * it + 2
        chunk(g, 0, True, True)
        chunk(g + 1, 1, True, True)
        return carry

    lax.fori_loop(0, (CPW - 4) // 2, step, 0)

    chunk(CPW - 2, 0, True, True)
    chunk(CPW - 1, 1, False, True)
    wait_store(CPW - 2, 0)
    wait_store(CPW - 1, 1)


@jax.jit
def _embed(xr, table):
    mesh = plsc.VectorSubcoreMesh(core_axis_name="c", subcore_axis_name="s")
    fn = pl.kernel(
        _body,
        out_type=jax.ShapeDtypeStruct((4 * HIST, CB, 8, LANE), jnp.float32),
        mesh=mesh,
        scratch_types=[
            pltpu.VMEM((TPW, 8, LANE), jnp.int32),
            pltpu.VMEM((4 * LANE, EMBED_DIM), jnp.float32),
            pltpu.VMEM((4 * LANE, EMBED_DIM), jnp.float32),
            pltpu.VMEM((16, 8, LANE), jnp.float32),
            pltpu.VMEM((16, 8, LANE), jnp.float32),
            pltpu.SemaphoreType.DMA,
            pltpu.SemaphoreType.DMA,
            pltpu.SemaphoreType.DMA,
            pltpu.SemaphoreType.DMA,
        ],
        compiler_params=pltpu.CompilerParams(use_tc_tiling_on_sc=False, needs_layout_passes=False),
    )
    return fn(xr, table)


def kernel(x, table):
    xr = (
        x.astype(jnp.int32)
        .T.reshape(H8, 8, CB, LANE)
        .transpose(0, 2, 1, 3)
        .reshape(TILES, 8, LANE)
    )
    out5 = _embed(xr, table)
    out = (
        out5.reshape(HIST, 4, CB, 8, LANE)
        .transpose(2, 4, 0, 1, 3)
        .reshape(BATCH, HIST, EMBED_DIM)
    )
    return out
